# baseline (speedup 1.0000x reference)
import functools

import jax
import jax.numpy as jnp
from jax import lax
from jax.experimental import pallas as pl
from jax.experimental.pallas import tpu as pltpu
from jax.experimental.pallas import tpu_sc as plsc

B, S = 4096, 200
N = B * S
V = 1_000_000
NC, NS = 2, 16
NW = NC * NS
N_PER_W = N // NW
STAGE = 62496
NSUB = 12
SUBSTAGE = STAGE // NSUB
STAGE_TAIL = V - NS * STAGE
KHBM = 9856
SEG = (0, 7808, 15744, 25600)

_mesh = plsc.VectorSubcoreMesh(core_axis_name="c", subcore_axis_name="s")


@functools.partial(
    pl.kernel,
    mesh=_mesh,
    out_type=jax.ShapeDtypeStruct((NW, N_PER_W), jnp.float32),
    scratch_types=[
        pltpu.VMEM_SHARED((V,), jnp.float32),
        pltpu.VMEM((N_PER_W,), jnp.int32),
        pltpu.VMEM((N_PER_W,), jnp.float32),
        pltpu.SemaphoreType.DMA,
        pltpu.SemaphoreType.DMA,
        pltpu.SemaphoreType.DMA,
    ],
)
def _sc_gather(table_hbm, idx_hbm, out_hbm, table_sp, idx_v, rows_v, sem, isem, ssem):
    cid = lax.axis_index("c")
    sid = lax.axis_index("s")
    wid = sid * NC + cid

    idx_d = pltpu.async_copy(idx_hbm.at[wid], idx_v, isem)

    base = sid * STAGE
    loads = [
        pltpu.async_copy(
            table_hbm.at[pl.ds(base + s * SUBSTAGE, SUBSTAGE)],
            rows_v.at[pl.ds((s % 2) * SUBSTAGE, SUBSTAGE)],
            ssem,
        )
        for s in range(2)
    ]
    stores = []
    for s in range(NSUB):
        loads[s].wait()
        stores.append(
            pltpu.async_copy(
                rows_v.at[pl.ds((s % 2) * SUBSTAGE, SUBSTAGE)],
                table_sp.at[pl.ds(base + s * SUBSTAGE, SUBSTAGE)],
                sem,
            )
        )
        if s + 2 < NSUB:
            stores[s].wait()
            loads.append(
                pltpu.async_copy(
                    table_hbm.at[pl.ds(base + (s + 2) * SUBSTAGE, SUBSTAGE)],
                    rows_v.at[pl.ds((s % 2) * SUBSTAGE, SUBSTAGE)],
                    ssem,
                )
            )
    for d in stores[-2:]:
        d.wait()

    @pl.when(sid == 0)
    def _():
        pltpu.sync_copy(
            table_hbm.at[pl.ds(NS * STAGE, STAGE_TAIL)],
            rows_v.at[pl.ds(0, STAGE_TAIL)],
        )
        pltpu.sync_copy(
            rows_v.at[pl.ds(0, STAGE_TAIL)],
            table_sp.at[pl.ds(NS * STAGE, STAGE_TAIL)],
        )

    idx_d.wait()
    hbm_d = pltpu.async_copy(
        table_hbm.at[idx_v.at[pl.ds(SEG[2], KHBM)]],
        rows_v.at[pl.ds(SEG[2], KHBM)],
        isem,
    )
    plsc.subcore_barrier()
    sp_d = [
        pltpu.async_copy(
            table_sp.at[idx_v.at[pl.ds(SEG[k], SEG[k + 1] - SEG[k])]],
            rows_v.at[pl.ds(SEG[k], SEG[k + 1] - SEG[k])],
            sem,
        )
        for k in range(2)
    ]
    wb = []
    for k in range(2):
        sp_d[k].wait()
        wb.append(
            pltpu.async_copy(
                rows_v.at[pl.ds(SEG[k], SEG[k + 1] - SEG[k])],
                out_hbm.at[wid, pl.ds(SEG[k], SEG[k + 1] - SEG[k])],
                ssem,
            )
        )
    hbm_d.wait()
    wb.append(
        pltpu.async_copy(
            rows_v.at[pl.ds(SEG[2], KHBM)],
            out_hbm.at[wid, pl.ds(SEG[2], KHBM)],
            ssem,
        )
    )
    for d in wb:
        d.wait()


def kernel(inputs, importance_embedding):
    idx = inputs.astype(jnp.int32).reshape(NW, N_PER_W)
    out = _sc_gather(importance_embedding, idx)
    return out.reshape(B, S)


# speedup vs baseline: 1.1100x; 1.1100x over previous
import functools

import jax
import jax.numpy as jnp
from jax import lax
from jax.experimental import pallas as pl
from jax.experimental.pallas import tpu as pltpu
from jax.experimental.pallas import tpu_sc as plsc

B, S = 4096, 200
N = B * S
V = 1_000_000
NC, NS = 2, 16
NW = NC * NS
N_PER_W = N // NW
STAGE = 62496
NSUB = 12
SUBSTAGE = STAGE // NSUB
STAGE_TAIL = V - NS * STAGE
KHBM = 4864
SEG = (0, 10368, 20736, 25600)

_mesh = plsc.VectorSubcoreMesh(core_axis_name="c", subcore_axis_name="s")


@functools.partial(
    pl.kernel,
    mesh=_mesh,
    out_type=jax.ShapeDtypeStruct((NW, N_PER_W), jnp.float32),
    scratch_types=[
        pltpu.VMEM_SHARED((V,), jnp.float32),
        pltpu.VMEM((N_PER_W,), jnp.int32),
        pltpu.VMEM((N_PER_W,), jnp.float32),
        pltpu.SemaphoreType.DMA,
        pltpu.SemaphoreType.DMA,
        pltpu.SemaphoreType.DMA,
    ],
)
def _sc_gather(table_hbm, idx_hbm, out_hbm, table_sp, idx_v, rows_v, sem, isem, ssem):
    cid = lax.axis_index("c")
    sid = lax.axis_index("s")
    wid = sid * NC + cid

    idx_d = pltpu.async_copy(idx_hbm.at[wid], idx_v, isem)

    base = sid * STAGE
    loads = [
        pltpu.async_copy(
            table_hbm.at[pl.ds(base + s * SUBSTAGE, SUBSTAGE)],
            rows_v.at[pl.ds((s % 2) * SUBSTAGE, SUBSTAGE)],
            ssem,
        )
        for s in range(2)
    ]
    stores = []
    for s in range(NSUB):
        loads[s].wait()
        stores.append(
            pltpu.async_copy(
                rows_v.at[pl.ds((s % 2) * SUBSTAGE, SUBSTAGE)],
                table_sp.at[pl.ds(base + s * SUBSTAGE, SUBSTAGE)],
                sem,
            )
        )
        if s + 2 < NSUB:
            stores[s].wait()
            loads.append(
                pltpu.async_copy(
                    table_hbm.at[pl.ds(base + (s + 2) * SUBSTAGE, SUBSTAGE)],
                    rows_v.at[pl.ds((s % 2) * SUBSTAGE, SUBSTAGE)],
                    ssem,
                )
            )
    for d in stores[-2:]:
        d.wait()

    @pl.when(sid == 0)
    def _():
        pltpu.sync_copy(
            table_hbm.at[pl.ds(NS * STAGE, STAGE_TAIL)],
            rows_v.at[pl.ds(0, STAGE_TAIL)],
        )
        pltpu.sync_copy(
            rows_v.at[pl.ds(0, STAGE_TAIL)],
            table_sp.at[pl.ds(NS * STAGE, STAGE_TAIL)],
        )

    idx_d.wait()
    hbm_d = pltpu.async_copy(
        table_hbm.at[idx_v.at[pl.ds(SEG[2], KHBM)]],
        rows_v.at[pl.ds(SEG[2], KHBM)],
        isem,
    )
    plsc.subcore_barrier()
    sp_d = [
        pltpu.async_copy(
            table_sp.at[idx_v.at[pl.ds(SEG[k], SEG[k + 1] - SEG[k])]],
            rows_v.at[pl.ds(SEG[k], SEG[k + 1] - SEG[k])],
            sem,
        )
        for k in range(2)
    ]
    wb = []
    for k in range(2):
        sp_d[k].wait()
        wb.append(
            pltpu.async_copy(
                rows_v.at[pl.ds(SEG[k], SEG[k + 1] - SEG[k])],
                out_hbm.at[wid, pl.ds(SEG[k], SEG[k + 1] - SEG[k])],
                ssem,
            )
        )
    hbm_d.wait()
    wb.append(
        pltpu.async_copy(
            rows_v.at[pl.ds(SEG[2], KHBM)],
            out_hbm.at[wid, pl.ds(SEG[2], KHBM)],
            ssem,
        )
    )
    for d in wb:
        d.wait()


def kernel(inputs, importance_embedding):
    idx = inputs.astype(jnp.int32).reshape(NW, N_PER_W)
    out = _sc_gather(importance_embedding, idx)
    return out.reshape(B, S)


# speedup vs baseline: 1.1580x; 1.0433x over previous
import functools

import jax
import jax.numpy as jnp
from jax import lax
from jax.experimental import pallas as pl
from jax.experimental.pallas import tpu as pltpu
from jax.experimental.pallas import tpu_sc as plsc

B, S = 4096, 200
N = B * S
V = 1_000_000
NC, NS = 2, 16
NW = NC * NS
N_PER_W = N // NW
STAGE = 62496
NSUB = 6
SUBSTAGE = STAGE // NSUB
STAGE_TAIL = V - NS * STAGE
KHBM = 4768
SEG = (0, 10416, 20832, 25600)

_mesh = plsc.VectorSubcoreMesh(core_axis_name="c", subcore_axis_name="s")


@functools.partial(
    pl.kernel,
    mesh=_mesh,
    out_type=jax.ShapeDtypeStruct((NW, N_PER_W), jnp.float32),
    scratch_types=[
        pltpu.VMEM_SHARED((V,), jnp.float32),
        pltpu.VMEM((N_PER_W,), jnp.int32),
        pltpu.VMEM((N_PER_W,), jnp.float32),
        pltpu.SemaphoreType.DMA,
        pltpu.SemaphoreType.DMA,
        pltpu.SemaphoreType.DMA,
    ],
)
def _sc_gather(table_hbm, idx_hbm, out_hbm, table_sp, idx_v, rows_v, sem, isem, ssem):
    cid = lax.axis_index("c")
    sid = lax.axis_index("s")
    wid = sid * NC + cid

    idx_d = pltpu.async_copy(idx_hbm.at[wid], idx_v, isem)

    base = sid * STAGE
    loads = [
        pltpu.async_copy(
            table_hbm.at[pl.ds(base + s * SUBSTAGE, SUBSTAGE)],
            rows_v.at[pl.ds((s % 2) * SUBSTAGE, SUBSTAGE)],
            ssem,
        )
        for s in range(2)
    ]
    stores = []
    for s in range(NSUB):
        loads[s].wait()
        stores.append(
            pltpu.async_copy(
                rows_v.at[pl.ds((s % 2) * SUBSTAGE, SUBSTAGE)],
                table_sp.at[pl.ds(base + s * SUBSTAGE, SUBSTAGE)],
                sem,
            )
        )
        if s + 2 < NSUB:
            stores[s].wait()
            loads.append(
                pltpu.async_copy(
                    table_hbm.at[pl.ds(base + (s + 2) * SUBSTAGE, SUBSTAGE)],
                    rows_v.at[pl.ds((s % 2) * SUBSTAGE, SUBSTAGE)],
                    ssem,
                )
            )
    for d in stores[-2:]:
        d.wait()

    @pl.when(sid == 0)
    def _():
        pltpu.sync_copy(
            table_hbm.at[pl.ds(NS * STAGE, STAGE_TAIL)],
            rows_v.at[pl.ds(0, STAGE_TAIL)],
        )
        pltpu.sync_copy(
            rows_v.at[pl.ds(0, STAGE_TAIL)],
            table_sp.at[pl.ds(NS * STAGE, STAGE_TAIL)],
        )

    idx_d.wait()
    hbm_d = pltpu.async_copy(
        table_hbm.at[idx_v.at[pl.ds(SEG[2], KHBM)]],
        rows_v.at[pl.ds(SEG[2], KHBM)],
        isem,
    )
    plsc.subcore_barrier()
    sp_d = [
        pltpu.async_copy(
            table_sp.at[idx_v.at[pl.ds(SEG[k], SEG[k + 1] - SEG[k])]],
            rows_v.at[pl.ds(SEG[k], SEG[k + 1] - SEG[k])],
            sem,
        )
        for k in range(2)
    ]
    for d in sp_d:
        d.wait()
    hbm_d.wait()
    pltpu.sync_copy(rows_v, out_hbm.at[wid])


def kernel(inputs, importance_embedding):
    idx = inputs.astype(jnp.int32).reshape(NW, N_PER_W)
    out = _sc_gather(importance_embedding, idx)
    return out.reshape(B, S)
